# baseline (speedup 1.0000x reference)
import jax
import jax.numpy as jnp
from jax.experimental import pallas as pl
from jax.experimental.pallas import tpu as pltpu

_IMG = 128
_NPIX = _IMG * _IMG
_K = 8
_SB = 8


def _topk_kernel(img_ref, pix_ref, tgt_ref):
    pix = pix_ref[0]
    lanes = jax.lax.broadcasted_iota(jnp.int32, (_SB, _NPIX), 1)
    eq = lanes == pix
    r0 = img_ref[0, 0][None, :]
    r1 = img_ref[0, 1][None, :]
    r2 = img_ref[0, 2][None, :]
    zero = jnp.zeros((_SB, _NPIX), jnp.float32)
    c0 = jnp.sum(jnp.where(eq, r0, zero), axis=1, keepdims=True)
    c1 = jnp.sum(jnp.where(eq, r1, zero), axis=1, keepdims=True)
    c2 = jnp.sum(jnp.where(eq, r2, zero), axis=1, keepdims=True)
    d = (jnp.abs(r0 - c0) + jnp.abs(r1 - c1) + jnp.abs(r2 - c2)) / 3.0
    picks = []
    for _ in range(_K):
        i = jnp.argmin(d, axis=1)
        picks.append(i)
        d = jnp.where(lanes == i[:, None], jnp.float32(jnp.inf), d)
    xs = [(p % _IMG).astype(jnp.float32) / _IMG for p in picks]
    ys = [(p // _IMG).astype(jnp.float32) / _IMG for p in picks]
    tgt_ref[0] = jnp.stack(xs + ys, axis=1)


def _loss_kernel(tgt_ref, pxn_ref, pyn_ref, out_ref):
    tx = tgt_ref[:, 0, 0:_K]
    ty = tgt_ref[:, 0, _K:2 * _K]
    dx = pxn_ref[:, 0:1] - tx
    dy = pyn_ref[:, 0:1] - ty
    dist = jnp.sqrt(dx * dx + dy * dy)
    mn = jnp.min(dist, axis=1)
    out_ref[:, :] = (jnp.sum(mn) / jnp.float32(4 * (_IMG - 1))).reshape(1, 1)


def kernel(predictions, ref_imgs):
    bs, L, _ = predictions.shape
    pos = predictions[:, :, :2]
    tmp = pos.reshape(bs * L, 2)
    q = tmp.reshape(L, bs, 2).transpose(1, 0, 2)
    gx = 2.0 * q[:, :, 0] - 1.0
    gy = 2.0 * q[:, :, 1] - 1.0
    fx = ((gx + 1.0) * _IMG - 1.0) / 2.0
    fy = ((gy + 1.0) * _IMG - 1.0) / 2.0
    ix_all = jnp.clip(jnp.round(fx), 0, _IMG - 1).astype(jnp.int32)
    iy_all = jnp.clip(jnp.round(fy), 0, _IMG - 1).astype(jnp.int32)
    pix = (iy_all * _IMG + ix_all).reshape(bs * L // _SB, _SB, 1)

    ngroups = L // _SB
    img_flat = ref_imgs.reshape(bs, 3, _NPIX)
    tgt = pl.pallas_call(
        _topk_kernel,
        grid=(bs, ngroups),
        in_specs=[
            pl.BlockSpec((1, 3, _NPIX), lambda b, g: (b, 0, 0)),
            pl.BlockSpec((1, _SB, 1), lambda b, g: (b * ngroups + g, 0, 0)),
        ],
        out_specs=pl.BlockSpec((1, _SB, 2 * _K),
                               lambda b, g: (b * ngroups + g, 0, 0)),
        out_shape=jax.ShapeDtypeStruct((bs * L // _SB, _SB, 2 * _K),
                                       jnp.float32),
        compiler_params=pltpu.CompilerParams(
            dimension_semantics=("parallel", "parallel")),
    )(img_flat, pix)

    n1 = bs * (L - 1)
    tgt_prev = tgt.reshape(bs, L, 2 * _K)[:, :L - 1].reshape(n1, 1, 2 * _K)
    pxn = predictions[:, 1:, 0].reshape(n1, 1)
    pyn = predictions[:, 1:, 1].reshape(n1, 1)
    res = pl.pallas_call(
        _loss_kernel,
        out_shape=jax.ShapeDtypeStruct((1, 1), jnp.float32),
    )(tgt_prev, pxn, pyn)
    return res[0, 0]


# speedup vs baseline: 1.5932x; 1.5932x over previous
import jax
import jax.numpy as jnp
from jax.experimental import pallas as pl
from jax.experimental.pallas import tpu as pltpu

_IMG = 128
_NPIX = _IMG * _IMG
_K = 8
_SB = 32


def _topk_kernel(img_ref, pix_ref, tgt_ref):
    pix = pix_ref[0]
    lanes = jax.lax.broadcasted_iota(jnp.int32, (_SB, _NPIX), 1)
    eq = lanes == pix
    r0 = img_ref[0, 0][None, :]
    r1 = img_ref[0, 1][None, :]
    r2 = img_ref[0, 2][None, :]
    zero = jnp.zeros((_SB, _NPIX), jnp.float32)
    c0 = jnp.sum(jnp.where(eq, r0, zero), axis=1, keepdims=True)
    c1 = jnp.sum(jnp.where(eq, r1, zero), axis=1, keepdims=True)
    c2 = jnp.sum(jnp.where(eq, r2, zero), axis=1, keepdims=True)
    d = (jnp.abs(r0 - c0) + jnp.abs(r1 - c1) + jnp.abs(r2 - c2)) / 3.0
    picks = []
    for _ in range(_K):
        i = jnp.argmin(d, axis=1)
        picks.append(i)
        d = jnp.where(lanes == i[:, None], jnp.float32(jnp.inf), d)
    xs = [(p % _IMG).astype(jnp.float32) / _IMG for p in picks]
    ys = [(p // _IMG).astype(jnp.float32) / _IMG for p in picks]
    tgt_ref[0] = jnp.stack(xs + ys, axis=1)


def _loss_kernel(tgt_ref, pxn_ref, pyn_ref, out_ref):
    tx = tgt_ref[:, 0, 0:_K]
    ty = tgt_ref[:, 0, _K:2 * _K]
    dx = pxn_ref[:, 0:1] - tx
    dy = pyn_ref[:, 0:1] - ty
    dist = jnp.sqrt(dx * dx + dy * dy)
    mn = jnp.min(dist, axis=1)
    out_ref[:, :] = (jnp.sum(mn) / jnp.float32(4 * (_IMG - 1))).reshape(1, 1)


def kernel(predictions, ref_imgs):
    bs, L, _ = predictions.shape
    pos = predictions[:, :, :2]
    tmp = pos.reshape(bs * L, 2)
    q = tmp.reshape(L, bs, 2).transpose(1, 0, 2)
    gx = 2.0 * q[:, :, 0] - 1.0
    gy = 2.0 * q[:, :, 1] - 1.0
    fx = ((gx + 1.0) * _IMG - 1.0) / 2.0
    fy = ((gy + 1.0) * _IMG - 1.0) / 2.0
    ix_all = jnp.clip(jnp.round(fx), 0, _IMG - 1).astype(jnp.int32)
    iy_all = jnp.clip(jnp.round(fy), 0, _IMG - 1).astype(jnp.int32)
    pix = (iy_all * _IMG + ix_all).reshape(bs * L // _SB, _SB, 1)

    ngroups = L // _SB
    img_flat = ref_imgs.reshape(bs, 3, _NPIX)
    tgt = pl.pallas_call(
        _topk_kernel,
        grid=(bs, ngroups),
        in_specs=[
            pl.BlockSpec((1, 3, _NPIX), lambda b, g: (b, 0, 0)),
            pl.BlockSpec((1, _SB, 1), lambda b, g: (b * ngroups + g, 0, 0)),
        ],
        out_specs=pl.BlockSpec((1, _SB, 2 * _K),
                               lambda b, g: (b * ngroups + g, 0, 0)),
        out_shape=jax.ShapeDtypeStruct((bs * L // _SB, _SB, 2 * _K),
                                       jnp.float32),
        compiler_params=pltpu.CompilerParams(
            dimension_semantics=("parallel", "parallel")),
    )(img_flat, pix)

    n1 = bs * (L - 1)
    tgt_prev = tgt.reshape(bs, L, 2 * _K)[:, :L - 1].reshape(n1, 1, 2 * _K)
    pxn = predictions[:, 1:, 0].reshape(n1, 1)
    pyn = predictions[:, 1:, 1].reshape(n1, 1)
    res = pl.pallas_call(
        _loss_kernel,
        out_shape=jax.ShapeDtypeStruct((1, 1), jnp.float32),
    )(tgt_prev, pxn, pyn)
    return res[0, 0]


# speedup vs baseline: 1.6620x; 1.0432x over previous
import jax
import jax.numpy as jnp
from jax.experimental import pallas as pl
from jax.experimental.pallas import tpu as pltpu

_IMG = 128
_NPIX = _IMG * _IMG
_K = 8
_SB = 64


def _topk_kernel(img_ref, pix_ref, tgt_ref):
    pix = pix_ref[0]
    lanes = jax.lax.broadcasted_iota(jnp.int32, (_SB, _NPIX), 1)
    eq = lanes == pix
    r0 = img_ref[0, 0][None, :]
    r1 = img_ref[0, 1][None, :]
    r2 = img_ref[0, 2][None, :]
    zero = jnp.zeros((_SB, _NPIX), jnp.float32)
    c0 = jnp.sum(jnp.where(eq, r0, zero), axis=1, keepdims=True)
    c1 = jnp.sum(jnp.where(eq, r1, zero), axis=1, keepdims=True)
    c2 = jnp.sum(jnp.where(eq, r2, zero), axis=1, keepdims=True)
    d = (jnp.abs(r0 - c0) + jnp.abs(r1 - c1) + jnp.abs(r2 - c2)) / 3.0
    picks = []
    for _ in range(_K):
        i = jnp.argmin(d, axis=1)
        picks.append(i)
        d = jnp.where(lanes == i[:, None], jnp.float32(jnp.inf), d)
    xs = [(p % _IMG).astype(jnp.float32) / _IMG for p in picks]
    ys = [(p // _IMG).astype(jnp.float32) / _IMG for p in picks]
    tgt_ref[0] = jnp.stack(xs + ys, axis=1)


def _loss_kernel(tgt_ref, pxn_ref, pyn_ref, out_ref):
    tx = tgt_ref[:, 0, 0:_K]
    ty = tgt_ref[:, 0, _K:2 * _K]
    dx = pxn_ref[:, 0:1] - tx
    dy = pyn_ref[:, 0:1] - ty
    dist = jnp.sqrt(dx * dx + dy * dy)
    mn = jnp.min(dist, axis=1)
    out_ref[:, :] = (jnp.sum(mn) / jnp.float32(4 * (_IMG - 1))).reshape(1, 1)


def kernel(predictions, ref_imgs):
    bs, L, _ = predictions.shape
    pos = predictions[:, :, :2]
    tmp = pos.reshape(bs * L, 2)
    q = tmp.reshape(L, bs, 2).transpose(1, 0, 2)
    gx = 2.0 * q[:, :, 0] - 1.0
    gy = 2.0 * q[:, :, 1] - 1.0
    fx = ((gx + 1.0) * _IMG - 1.0) / 2.0
    fy = ((gy + 1.0) * _IMG - 1.0) / 2.0
    ix_all = jnp.clip(jnp.round(fx), 0, _IMG - 1).astype(jnp.int32)
    iy_all = jnp.clip(jnp.round(fy), 0, _IMG - 1).astype(jnp.int32)
    pix = (iy_all * _IMG + ix_all).reshape(bs * L // _SB, _SB, 1)

    ngroups = L // _SB
    img_flat = ref_imgs.reshape(bs, 3, _NPIX)
    tgt = pl.pallas_call(
        _topk_kernel,
        grid=(bs, ngroups),
        in_specs=[
            pl.BlockSpec((1, 3, _NPIX), lambda b, g: (b, 0, 0)),
            pl.BlockSpec((1, _SB, 1), lambda b, g: (b * ngroups + g, 0, 0)),
        ],
        out_specs=pl.BlockSpec((1, _SB, 2 * _K),
                               lambda b, g: (b * ngroups + g, 0, 0)),
        out_shape=jax.ShapeDtypeStruct((bs * L // _SB, _SB, 2 * _K),
                                       jnp.float32),
        compiler_params=pltpu.CompilerParams(
            dimension_semantics=("parallel", "parallel")),
    )(img_flat, pix)

    n1 = bs * (L - 1)
    tgt_prev = tgt.reshape(bs, L, 2 * _K)[:, :L - 1].reshape(n1, 1, 2 * _K)
    pxn = predictions[:, 1:, 0].reshape(n1, 1)
    pyn = predictions[:, 1:, 1].reshape(n1, 1)
    res = pl.pallas_call(
        _loss_kernel,
        out_shape=jax.ShapeDtypeStruct((1, 1), jnp.float32),
    )(tgt_prev, pxn, pyn)
    return res[0, 0]


# speedup vs baseline: 1.7156x; 1.0322x over previous
import jax
import jax.numpy as jnp
from jax.experimental import pallas as pl
from jax.experimental.pallas import tpu as pltpu

_IMG = 128
_NPIX = _IMG * _IMG
_K = 8
_SB = 128


def _topk_kernel(img_ref, pix_ref, tgt_ref):
    pix = pix_ref[0]
    lanes = jax.lax.broadcasted_iota(jnp.int32, (_SB, _NPIX), 1)
    eq = lanes == pix
    r0 = img_ref[0, 0][None, :]
    r1 = img_ref[0, 1][None, :]
    r2 = img_ref[0, 2][None, :]
    zero = jnp.zeros((_SB, _NPIX), jnp.float32)
    c0 = jnp.sum(jnp.where(eq, r0, zero), axis=1, keepdims=True)
    c1 = jnp.sum(jnp.where(eq, r1, zero), axis=1, keepdims=True)
    c2 = jnp.sum(jnp.where(eq, r2, zero), axis=1, keepdims=True)
    d = (jnp.abs(r0 - c0) + jnp.abs(r1 - c1) + jnp.abs(r2 - c2)) / 3.0
    picks = []
    for _ in range(_K):
        i = jnp.argmin(d, axis=1)
        picks.append(i)
        d = jnp.where(lanes == i[:, None], jnp.float32(jnp.inf), d)
    xs = [(p % _IMG).astype(jnp.float32) / _IMG for p in picks]
    ys = [(p // _IMG).astype(jnp.float32) / _IMG for p in picks]
    tgt_ref[0] = jnp.stack(xs + ys, axis=1)


def _loss_kernel(tgt_ref, pxn_ref, pyn_ref, out_ref):
    tx = tgt_ref[:, 0, 0:_K]
    ty = tgt_ref[:, 0, _K:2 * _K]
    dx = pxn_ref[:, 0:1] - tx
    dy = pyn_ref[:, 0:1] - ty
    dist = jnp.sqrt(dx * dx + dy * dy)
    mn = jnp.min(dist, axis=1)
    out_ref[:, :] = (jnp.sum(mn) / jnp.float32(4 * (_IMG - 1))).reshape(1, 1)


def kernel(predictions, ref_imgs):
    bs, L, _ = predictions.shape
    pos = predictions[:, :, :2]
    tmp = pos.reshape(bs * L, 2)
    q = tmp.reshape(L, bs, 2).transpose(1, 0, 2)
    gx = 2.0 * q[:, :, 0] - 1.0
    gy = 2.0 * q[:, :, 1] - 1.0
    fx = ((gx + 1.0) * _IMG - 1.0) / 2.0
    fy = ((gy + 1.0) * _IMG - 1.0) / 2.0
    ix_all = jnp.clip(jnp.round(fx), 0, _IMG - 1).astype(jnp.int32)
    iy_all = jnp.clip(jnp.round(fy), 0, _IMG - 1).astype(jnp.int32)
    pix = (iy_all * _IMG + ix_all).reshape(bs * L // _SB, _SB, 1)

    ngroups = L // _SB
    img_flat = ref_imgs.reshape(bs, 3, _NPIX)
    tgt = pl.pallas_call(
        _topk_kernel,
        grid=(bs, ngroups),
        in_specs=[
            pl.BlockSpec((1, 3, _NPIX), lambda b, g: (b, 0, 0)),
            pl.BlockSpec((1, _SB, 1), lambda b, g: (b * ngroups + g, 0, 0)),
        ],
        out_specs=pl.BlockSpec((1, _SB, 2 * _K),
                               lambda b, g: (b * ngroups + g, 0, 0)),
        out_shape=jax.ShapeDtypeStruct((bs * L // _SB, _SB, 2 * _K),
                                       jnp.float32),
        compiler_params=pltpu.CompilerParams(
            dimension_semantics=("parallel", "parallel")),
    )(img_flat, pix)

    n1 = bs * (L - 1)
    tgt_prev = tgt.reshape(bs, L, 2 * _K)[:, :L - 1].reshape(n1, 1, 2 * _K)
    pxn = predictions[:, 1:, 0].reshape(n1, 1)
    pyn = predictions[:, 1:, 1].reshape(n1, 1)
    res = pl.pallas_call(
        _loss_kernel,
        out_shape=jax.ShapeDtypeStruct((1, 1), jnp.float32),
    )(tgt_prev, pxn, pyn)
    return res[0, 0]
